# Initial kernel scaffold; baseline (speedup 1.0000x reference)
import jax
import jax.numpy as jnp
from jax.experimental import pallas as pl


def kernel(x, edge_index, edge_attr, W1, b1, W2, b2):
    raise NotImplementedError("write your pallas kernel here")


# speedup vs baseline: 64.7534x; 64.7534x over previous
import functools

import jax
import jax.numpy as jnp
from jax import lax
from jax.experimental import pallas as pl
from jax.experimental.pallas import tpu as pltpu
from jax.experimental.pallas import tpu_sc as plsc

N = 50000
E = 800000
F_IN = 4
H = 64

NP = 50176
EP = 819200
NC = 2
NS = 16
NW = NC * NS
PT = EP // NW
CH = 12800
K = PT // CH
ZS = NP // NS

_mesh = plsc.VectorSubcoreMesh(core_axis_name="c", subcore_axis_name="s")


def _zero_vmem(buf, n):
    z = jnp.zeros((16,), jnp.float32)

    def body(i, carry):
        buf[pl.ds(pl.multiple_of(i * 16, 16), 16)] = z
        return carry

    lax.fori_loop(0, n // 16, body, 0)


@functools.partial(
    pl.kernel,
    out_type=jax.ShapeDtypeStruct((NC * NP,), jnp.float32),
    mesh=_mesh,
    scratch_types=[
        pltpu.VMEM((ZS,), jnp.float32),
        pltpu.VMEM((CH,), jnp.int32),
        pltpu.VMEM((CH,), jnp.float32),
        pltpu.VMEM_SHARED((NP,), jnp.float32),
    ],
)
def _deg_sc(dst_hbm, w_hbm, out_hbm, zb, di, wv, acc_sp):
    c = lax.axis_index("c")
    s = lax.axis_index("s")
    sl = pl.ds(s * ZS, ZS)
    _zero_vmem(zb, ZS)
    pltpu.sync_copy(zb, acc_sp.at[sl])
    plsc.subcore_barrier()
    base = c * (EP // NC) + s * PT
    for k in range(K):
        off = base + k * CH
        pltpu.sync_copy(dst_hbm.at[pl.ds(off, CH)], di)
        pltpu.sync_copy(w_hbm.at[pl.ds(off, CH)], wv)
        pltpu.sync_copy(wv, acc_sp.at[di], add=True)
    plsc.subcore_barrier()
    pltpu.sync_copy(acc_sp.at[sl], zb)
    pltpu.sync_copy(zb, out_hbm.at[pl.ds(c * NP + s * ZS, ZS)])


@functools.partial(
    pl.kernel,
    out_type=tuple(
        jax.ShapeDtypeStruct((NC * NP,), jnp.float32) for _ in range(F_IN)
    ),
    mesh=_mesh,
    scratch_types=(
        [pltpu.VMEM((ZS,), jnp.float32),
         pltpu.VMEM((CH,), jnp.int32),
         pltpu.VMEM((CH,), jnp.int32),
         pltpu.VMEM((CH,), jnp.float32)]
        + [pltpu.VMEM((CH,), jnp.float32) for _ in range(F_IN)]
        + [pltpu.VMEM_SHARED((NP,), jnp.float32) for _ in range(F_IN)]
        + [pltpu.VMEM_SHARED((NP,), jnp.float32) for _ in range(F_IN)]
    ),
)
def _agg_sc(src_hbm, dst_hbm, w_hbm, xs0, xs1, xs2, xs3,
            o0, o1, o2, o3,
            zb, si, di, wv, g0, g1, g2, g3,
            t0, t1, t2, t3, a0, a1, a2, a3):
    xs_hbm = (xs0, xs1, xs2, xs3)
    outs = (o0, o1, o2, o3)
    gs = (g0, g1, g2, g3)
    ts = (t0, t1, t2, t3)
    accs = (a0, a1, a2, a3)
    c = lax.axis_index("c")
    s = lax.axis_index("s")
    sl = pl.ds(s * ZS, ZS)
    _zero_vmem(zb, ZS)
    for f in range(F_IN):
        pltpu.sync_copy(xs_hbm[f].at[sl], gs[f].at[pl.ds(0, ZS)])
        pltpu.sync_copy(gs[f].at[pl.ds(0, ZS)], ts[f].at[sl])
        pltpu.sync_copy(zb, accs[f].at[sl])
    plsc.subcore_barrier()
    base = c * (EP // NC) + s * PT
    for k in range(K):
        off = base + k * CH
        pltpu.sync_copy(src_hbm.at[pl.ds(off, CH)], si)
        pltpu.sync_copy(dst_hbm.at[pl.ds(off, CH)], di)
        pltpu.sync_copy(w_hbm.at[pl.ds(off, CH)], wv)
        for f in range(F_IN):
            pltpu.sync_copy(ts[f].at[si], gs[f])

        def mul_body(i, carry):
            d16 = pl.ds(pl.multiple_of(i * 16, 16), 16)
            wvec = wv[d16]
            g0[d16] = g0[d16] * wvec
            g1[d16] = g1[d16] * wvec
            g2[d16] = g2[d16] * wvec
            g3[d16] = g3[d16] * wvec
            return carry

        lax.fori_loop(0, CH // 16, mul_body, 0)
        for f in range(F_IN):
            pltpu.sync_copy(gs[f], accs[f].at[di], add=True)
    plsc.subcore_barrier()
    osl = pl.ds(c * NP + s * ZS, ZS)
    for f in range(F_IN):
        pltpu.sync_copy(accs[f].at[sl], zb)
        pltpu.sync_copy(zb, outs[f].at[osl])


@functools.partial(
    pl.kernel,
    out_type=jax.ShapeDtypeStruct((NC * NP,), jnp.float32),
    mesh=_mesh,
    scratch_types=[
        pltpu.VMEM((ZS,), jnp.float32),
        pltpu.VMEM((CH,), jnp.int32),
        pltpu.VMEM((CH,), jnp.int32),
        pltpu.VMEM((CH,), jnp.float32),
        pltpu.VMEM((CH,), jnp.float32),
        pltpu.VMEM_SHARED((NP,), jnp.float32),
        pltpu.VMEM_SHARED((NP,), jnp.float32),
    ],
)
def _out_sc(src_hbm, dst_hbm, w_hbm, ys_hbm, out_hbm,
            zb, si, di, wv, gv, ys_sp, acc_sp):
    c = lax.axis_index("c")
    s = lax.axis_index("s")
    sl = pl.ds(s * ZS, ZS)
    _zero_vmem(zb, ZS)
    pltpu.sync_copy(ys_hbm.at[sl], gv.at[pl.ds(0, ZS)])
    pltpu.sync_copy(gv.at[pl.ds(0, ZS)], ys_sp.at[sl])
    pltpu.sync_copy(zb, acc_sp.at[sl])
    plsc.subcore_barrier()
    base = c * (EP // NC) + s * PT
    for k in range(K):
        off = base + k * CH
        pltpu.sync_copy(src_hbm.at[pl.ds(off, CH)], si)
        pltpu.sync_copy(dst_hbm.at[pl.ds(off, CH)], di)
        pltpu.sync_copy(w_hbm.at[pl.ds(off, CH)], wv)
        pltpu.sync_copy(ys_sp.at[si], gv)

        def mul_body(i, carry):
            d16 = pl.ds(pl.multiple_of(i * 16, 16), 16)
            gv[d16] = gv[d16] * wv[d16]
            return carry

        lax.fori_loop(0, CH // 16, mul_body, 0)
        pltpu.sync_copy(gv, acc_sp.at[di], add=True)
    plsc.subcore_barrier()
    pltpu.sync_copy(acc_sp.at[sl], zb)
    pltpu.sync_copy(zb, out_hbm.at[pl.ds(c * NP + s * ZS, ZS)])


def _p2_body(degp_ref, xt_ref, dis_ref, selfn_ref, xst_ref):
    deg = degp_ref[0:1, :] + degp_ref[1:2, :] + 1.0
    dis = lax.rsqrt(deg)
    dis_ref[...] = dis
    selfn_ref[...] = dis * dis
    xst_ref[...] = xt_ref[...] * dis


_p2_tc = pl.pallas_call(
    _p2_body,
    out_shape=(
        jax.ShapeDtypeStruct((1, NP), jnp.float32),
        jax.ShapeDtypeStruct((1, NP), jnp.float32),
        jax.ShapeDtypeStruct((F_IN, NP), jnp.float32),
    ),
)


_B4 = NP // 8


def _p4_body(ag0, ag1, ag2, ag3, xt_ref, dis_ref, selfn_ref,
             w1t_ref, b1c_ref, w2_ref, b2s_ref, ys_ref, oinit_ref):
    dis = dis_ref[...]
    selfn = selfn_ref[...]
    xt = xt_ref[...]
    aggs = []
    for f, ag in enumerate((ag0, ag1, ag2, ag3)):
        aggs.append(dis * (ag[0:1, :] + ag[1:2, :])
                    + selfn * xt[f:f + 1, :])
    w1t = w1t_ref[...]
    h = b1c_ref[...]
    for f in range(F_IN):
        h = h + w1t[:, f:f + 1] * aggs[f]
    h = jnp.maximum(h, 0.0)
    y = jnp.sum(h * w2_ref[...], axis=0, keepdims=True)
    ys_ref[...] = y * dis
    oinit_ref[...] = y * selfn + b2s_ref[...]


_p4_tc = pl.pallas_call(
    _p4_body,
    grid=(NP // _B4,),
    in_specs=[
        pl.BlockSpec((NC, _B4), lambda i: (0, i)),
        pl.BlockSpec((NC, _B4), lambda i: (0, i)),
        pl.BlockSpec((NC, _B4), lambda i: (0, i)),
        pl.BlockSpec((NC, _B4), lambda i: (0, i)),
        pl.BlockSpec((F_IN, _B4), lambda i: (0, i)),
        pl.BlockSpec((1, _B4), lambda i: (0, i)),
        pl.BlockSpec((1, _B4), lambda i: (0, i)),
        pl.BlockSpec((H, F_IN), lambda i: (0, 0)),
        pl.BlockSpec((H, 1), lambda i: (0, 0)),
        pl.BlockSpec((H, 1), lambda i: (0, 0)),
        pl.BlockSpec((1, 1), lambda i: (0, 0)),
    ],
    out_specs=(
        pl.BlockSpec((1, _B4), lambda i: (0, i)),
        pl.BlockSpec((1, _B4), lambda i: (0, i)),
    ),
    out_shape=(
        jax.ShapeDtypeStruct((1, NP), jnp.float32),
        jax.ShapeDtypeStruct((1, NP), jnp.float32),
    ),
)


def _p6_body(outp_ref, dis_ref, oinit_ref, out_ref):
    out_ref[...] = (oinit_ref[...]
                    + dis_ref[...] * (outp_ref[0:1, :] + outp_ref[1:2, :]))


_p6_tc = pl.pallas_call(
    _p6_body,
    out_shape=jax.ShapeDtypeStruct((1, NP), jnp.float32),
)


def kernel(x, edge_index, edge_attr, W1, b1, W2, b2):
    f32 = jnp.float32
    pad_e = EP - E
    srcp = jnp.concatenate(
        [edge_index[0], jnp.full((pad_e,), N, dtype=jnp.int32)])
    dstp = jnp.concatenate(
        [edge_index[1], jnp.full((pad_e,), N, dtype=jnp.int32)])
    wp = jnp.concatenate([edge_attr.astype(f32), jnp.zeros((pad_e,), f32)])
    xt = jnp.pad(x.astype(f32).T, ((0, 0), (0, NP - N)))

    degp = _deg_sc(dstp, wp).reshape(NC, NP)
    dis, selfn, xst = _p2_tc(degp, xt)
    aggp = _agg_sc(srcp, dstp, wp, xst[0], xst[1], xst[2], xst[3])
    aggp = [a.reshape(NC, NP) for a in aggp]
    ys, oinit = _p4_tc(aggp[0], aggp[1], aggp[2], aggp[3], xt, dis, selfn,
                       W1.astype(f32).T, b1.astype(f32).reshape(H, 1),
                       W2.astype(f32), b2.astype(f32).reshape(1, 1))
    outp = _out_sc(srcp, dstp, wp, ys[0]).reshape(NC, NP)
    out = _p6_tc(outp, dis, oinit)
    return out[0, :N].reshape(N, 1)
